# baseline (speedup 1.0000x reference)
import functools

import jax
import jax.numpy as jnp
from jax import lax
from jax.experimental import pallas as pl
from jax.experimental.pallas import tpu as pltpu
from jax.experimental.pallas import tpu_sc as plsc

B = 1024
D = 128
LANES = 16
NC = 2
NS = 16
NW = NC * NS
BPW = B // NW


def _sc_gather_dot(i32, j32, v_w, w_w, bv, bw):
    mesh = plsc.VectorSubcoreMesh(
        core_axis_name="c", subcore_axis_name="s", num_cores=NC, num_subcores=NS
    )

    @functools.partial(
        pl.kernel,
        mesh=mesh,
        compiler_params=pltpu.CompilerParams(needs_layout_passes=False),
        out_type=[
            jax.ShapeDtypeStruct((B,), jnp.float32),
            jax.ShapeDtypeStruct((B,), jnp.float32),
        ],
        scratch_types=[
            pltpu.VMEM((BPW,), jnp.int32),
            pltpu.VMEM((BPW,), jnp.int32),
            pltpu.SMEM((BPW,), jnp.int32),
            pltpu.SMEM((BPW,), jnp.int32),
            pltpu.VMEM((BPW, D), jnp.float32),
            pltpu.VMEM((BPW, D), jnp.float32),
            pltpu.VMEM((BPW, 1), jnp.float32),
            pltpu.VMEM((BPW, 1), jnp.float32),
            pltpu.VMEM((BPW,), jnp.float32),
            pltpu.VMEM((BPW,), jnp.float32),
            pltpu.SemaphoreType.DMA,
            pltpu.SemaphoreType.DMA,
        ],
    )
    def sc_k(i_hbm, j_hbm, v_hbm, w_hbm, bv_hbm, bw_hbm,
             sim_hbm, bsum_hbm,
             iv, jv, ism, jsm, vrows, wrows, biv, bjv, simv, bsumv, sem, bsem):
        wid = lax.axis_index("s") * NC + lax.axis_index("c")
        base = wid * BPW
        pltpu.sync_copy(i_hbm.at[pl.ds(base, BPW)], iv)
        pltpu.sync_copy(j_hbm.at[pl.ds(base, BPW)], jv)
        for g in range(BPW // LANES):
            ivec = iv[pl.ds(g * LANES, LANES)]
            jvec = jv[pl.ds(g * LANES, LANES)]
            for l in range(LANES):
                ism[g * LANES + l] = ivec[l]
                jsm[g * LANES + l] = jvec[l]
        cps = [
            pltpu.async_copy(v_hbm.at[iv], vrows, sem),
            pltpu.async_copy(w_hbm.at[jv], wrows, sem),
        ]
        bcps = []
        for e in range(BPW):
            bcps.append(pltpu.async_copy(
                bv_hbm.at[pl.ds(ism[e], 1), :], biv.at[pl.ds(e, 1), :], bsem))
            bcps.append(pltpu.async_copy(
                bw_hbm.at[pl.ds(jsm[e], 1), :], bjv.at[pl.ds(e, 1), :], bsem))
        for cp in cps:
            cp.wait()
        lanes = lax.iota(jnp.int32, LANES)
        for g in range(BPW // LANES):
            sl = pl.ds(g * LANES, LANES)
            accs = [None] * LANES
            for k in range(D // LANES):
                ck = pl.ds(k * LANES, LANES)
                for l in range(LANES):
                    e = g * LANES + l
                    p = vrows[e, ck] * wrows[e, ck]
                    accs[l] = p if accs[l] is None else accs[l] + p
            sims = jnp.zeros((LANES,), jnp.float32)
            for l in range(LANES):
                sims = jnp.where(lanes == l, jnp.sum(accs[l]), sims)
            simv[sl] = sims
        for cp in bcps:
            cp.wait()
        zero16 = jnp.zeros((LANES,), jnp.int32)
        for g in range(BPW // LANES):
            sl = pl.ds(g * LANES, LANES)
            rows = lanes + g * LANES
            bsumv[sl] = (plsc.load_gather(biv, [rows, zero16])
                         + plsc.load_gather(bjv, [rows, zero16]))
        pltpu.sync_copy(simv, sim_hbm.at[pl.ds(base, BPW)])
        pltpu.sync_copy(bsumv, bsum_hbm.at[pl.ds(base, BPW)])

    return sc_k(i32, j32, v_w, w_w, bv, bw)


def _tc_finish_body(sim_ref, bsum_ref, co_ref, w_ref, out_ref):
    b = bsum_ref[...]
    mb = jnp.sum(b) * (1.0 / B)
    d = b - mb
    varb = jnp.sum(d * d)
    a = sim_ref[...] - jnp.log(co_ref[...]) + mb
    wv = w_ref[...]
    out_ref[0, 0] = 0.5 * (B * jnp.sum(wv * a * a) + varb * jnp.sum(wv))


def _tc_finish(sim, bsum, co, w):
    out = pl.pallas_call(
        _tc_finish_body,
        out_shape=jax.ShapeDtypeStruct((1, 1), jnp.float32),
        out_specs=pl.BlockSpec(memory_space=pltpu.SMEM),
    )(sim.reshape(8, B // 8), bsum.reshape(8, B // 8), co.reshape(8, B // 8),
      w.reshape(8, B // 8))
    return out[0, 0]


def kernel(i, j, co_occur, weight, v_weight, w_weight, biasv_weight, biasw_weight):
    sim, bsum = _sc_gather_dot(
        i.astype(jnp.int32), j.astype(jnp.int32),
        v_weight, w_weight, biasv_weight, biasw_weight)
    return _tc_finish(sim, bsum, co_occur, weight)


# speedup vs baseline: 2.5951x; 2.5951x over previous
import functools

import jax
import jax.numpy as jnp
from jax import lax
from jax.experimental import pallas as pl
from jax.experimental.pallas import tpu as pltpu
from jax.experimental.pallas import tpu_sc as plsc

B = 1024
D = 128
LANES = 16
NC = 2
NS = 16
NW = NC * NS
BPW = B // NW


def _sc_gather_dot(i32, j32, v_w, w_w, bvp, bwp):
    mesh = plsc.VectorSubcoreMesh(
        core_axis_name="c", subcore_axis_name="s", num_cores=NC, num_subcores=NS
    )

    @functools.partial(
        pl.kernel,
        mesh=mesh,
        compiler_params=pltpu.CompilerParams(needs_layout_passes=False),
        out_type=[
            jax.ShapeDtypeStruct((B,), jnp.float32),
            jax.ShapeDtypeStruct((B,), jnp.float32),
        ],
        scratch_types=[
            pltpu.VMEM((BPW,), jnp.int32),
            pltpu.VMEM((BPW,), jnp.int32),
            pltpu.VMEM((BPW,), jnp.int32),
            pltpu.VMEM((BPW,), jnp.int32),
            pltpu.VMEM((BPW, D), jnp.float32),
            pltpu.VMEM((BPW, D), jnp.float32),
            pltpu.VMEM((BPW, D), jnp.float32),
            pltpu.VMEM((BPW, D), jnp.float32),
            pltpu.VMEM((BPW,), jnp.float32),
            pltpu.VMEM((BPW,), jnp.float32),
            pltpu.SemaphoreType.DMA,
        ],
    )
    def sc_k(i_hbm, j_hbm, v_hbm, w_hbm, bv_hbm, bw_hbm,
             sim_hbm, bsum_hbm,
             iv, jv, ivh, jvh, vrows, wrows, bvrows, bwrows, simv, bsumv, sem):
        wid = lax.axis_index("s") * NC + lax.axis_index("c")
        base = wid * BPW
        pltpu.sync_copy(i_hbm.at[pl.ds(base, BPW)], iv)
        pltpu.sync_copy(j_hbm.at[pl.ds(base, BPW)], jv)
        for g in range(BPW // LANES):
            sl = pl.ds(g * LANES, LANES)
            ivh[sl] = iv[sl] >> 7
            jvh[sl] = jv[sl] >> 7
        cps = [
            pltpu.async_copy(v_hbm.at[iv], vrows, sem),
            pltpu.async_copy(w_hbm.at[jv], wrows, sem),
            pltpu.async_copy(bv_hbm.at[ivh], bvrows, sem),
            pltpu.async_copy(bw_hbm.at[jvh], bwrows, sem),
        ]
        for cp in cps:
            cp.wait()
        lanes = lax.iota(jnp.int32, LANES)
        for g in range(BPW // LANES):
            sl = pl.ds(g * LANES, LANES)
            sims = jnp.zeros((LANES,), jnp.float32)
            for l in range(LANES):
                e = g * LANES + l
                nk = D // LANES
                acc0 = vrows[e, 0:LANES] * wrows[e, 0:LANES]
                ck = pl.ds(LANES, LANES)
                acc1 = vrows[e, ck] * wrows[e, ck]
                for k in range(2, nk, 2):
                    c0 = pl.ds(k * LANES, LANES)
                    c1 = pl.ds((k + 1) * LANES, LANES)
                    acc0 = acc0 + vrows[e, c0] * wrows[e, c0]
                    acc1 = acc1 + vrows[e, c1] * wrows[e, c1]
                sims = jnp.where(lanes == l, jnp.sum(acc0 + acc1), sims)
            simv[sl] = sims
            rows = lanes + g * LANES
            bsumv[sl] = (plsc.load_gather(bvrows, [rows, iv[sl] & 127])
                         + plsc.load_gather(bwrows, [rows, jv[sl] & 127]))
        pltpu.sync_copy(simv, sim_hbm.at[pl.ds(base, BPW)])
        pltpu.sync_copy(bsumv, bsum_hbm.at[pl.ds(base, BPW)])

    return sc_k(i32, j32, v_w, w_w, bvp, bwp)


def _tc_finish_body(sim_ref, bsum_ref, co_ref, w_ref, out_ref):
    b = bsum_ref[...]
    mb = jnp.sum(b) * (1.0 / B)
    d = b - mb
    varb = jnp.sum(d * d)
    a = sim_ref[...] - jnp.log(co_ref[...]) + mb
    wv = w_ref[...]
    out_ref[0, 0] = 0.5 * (B * jnp.sum(wv * a * a) + varb * jnp.sum(wv))


def _tc_finish(sim, bsum, co, w):
    out = pl.pallas_call(
        _tc_finish_body,
        out_shape=jax.ShapeDtypeStruct((1, 1), jnp.float32),
        out_specs=pl.BlockSpec(memory_space=pltpu.SMEM),
    )(sim.reshape(8, B // 8), bsum.reshape(8, B // 8), co.reshape(8, B // 8),
      w.reshape(8, B // 8))
    return out[0, 0]


def _pad_bias(bias):
    flat = bias.reshape(-1)
    pad = (-flat.shape[0]) % D
    return jnp.pad(flat, (0, pad)).reshape(-1, D)


def kernel(i, j, co_occur, weight, v_weight, w_weight, biasv_weight, biasw_weight):
    sim, bsum = _sc_gather_dot(
        i.astype(jnp.int32), j.astype(jnp.int32),
        v_weight, w_weight, _pad_bias(biasv_weight), _pad_bias(biasw_weight))
    return _tc_finish(sim, bsum, co_occur, weight)
